# baseline (speedup 1.0000x reference)
import functools

import jax
import jax.numpy as jnp
from jax import lax
from jax.experimental import pallas as pl
from jax.experimental.pallas import tpu as pltpu
from jax.experimental.pallas import tpu_sc as plsc

NUM_TASKS = 100000
EMBED_DIM = 128
BATCH = 16384

_NC = 2
_NS = 16
_NW = _NC * _NS
_B_PER_W = BATCH // _NW


@functools.lru_cache(maxsize=4)
def _make_sc_gather(nrows):
    b_per_w = nrows // _NW
    mesh = plsc.VectorSubcoreMesh(core_axis_name="c", subcore_axis_name="s")

    half = b_per_w // 2

    @functools.partial(
        pl.kernel,
        mesh=mesh,
        out_type=jax.ShapeDtypeStruct((nrows, EMBED_DIM), jnp.float32),
        scratch_types=[
            pltpu.VMEM((b_per_w,), jnp.int32),
            pltpu.VMEM((half, EMBED_DIM), jnp.float32),
            pltpu.VMEM((half, EMBED_DIM), jnp.float32),
            pltpu.SemaphoreType.DMA,
            pltpu.SemaphoreType.DMA,
            pltpu.SemaphoreType.DMA,
            pltpu.SemaphoreType.DMA,
        ],
    )
    def sc_gather(table_hbm, idx_hbm, out_hbm, idx_v, rows_v0, rows_v1,
                  gsem0, gsem1, wsem0, wsem1):
        wid = lax.axis_index("s") * _NC + lax.axis_index("c")
        base = wid * b_per_w
        pltpu.sync_copy(idx_hbm.at[pl.ds(base, b_per_w)], idx_v)
        g0 = pltpu.async_copy(table_hbm.at[idx_v.at[pl.ds(0, half)]],
                              rows_v0, gsem0)
        g1 = pltpu.async_copy(table_hbm.at[idx_v.at[pl.ds(half, half)]],
                              rows_v1, gsem1)
        g0.wait()
        w0 = pltpu.async_copy(rows_v0, out_hbm.at[pl.ds(base, half)], wsem0)
        g1.wait()
        w1 = pltpu.async_copy(rows_v1, out_hbm.at[pl.ds(base + half, half)],
                              wsem1)
        w0.wait()
        w1.wait()

    return sc_gather


_ROW_BLK = 4096


def _mlp_body(e_ref, a_ref, b1c_ref, gamma_ref, beta_ref, w2t_ref, b2_ref,
              out_ref):
    e = e_ref[...]
    hc = lax.dot_general(e, a_ref[...], (((1,), (0,)), ((), ())),
                         preferred_element_type=jnp.float32)
    hc = hc + b1c_ref[...]
    avg = jnp.full((EMBED_DIM, EMBED_DIM), 1.0 / EMBED_DIM, dtype=jnp.float32)
    var = lax.dot_general(hc * hc, avg, (((1,), (0,)), ((), ())),
                          preferred_element_type=jnp.float32)
    h = hc * lax.rsqrt(var + 1e-5) * gamma_ref[...] + beta_ref[...]
    h = 0.5 * h * (1.0 + lax.erf(h * 0.7071067811865476))
    out = lax.dot_general(h, w2t_ref[...], (((1,), (0,)), ((), ())),
                          preferred_element_type=jnp.float32)
    out_ref[...] = out + b2_ref[...]


def _mlp(e, W1, b1, gamma, beta, W2, b2):
    A = W1.T - jnp.mean(W1, axis=0)[:, None]
    b1c = b1 - jnp.mean(b1)
    nrows = e.shape[0]
    grid = (nrows // _ROW_BLK,)
    full = pl.BlockSpec((EMBED_DIM, EMBED_DIM), lambda i: (0, 0))
    vec = pl.BlockSpec((1, EMBED_DIM), lambda i: (0, 0))
    return pl.pallas_call(
        _mlp_body,
        grid=grid,
        in_specs=[
            pl.BlockSpec((_ROW_BLK, EMBED_DIM), lambda i: (i, 0)),
            full, vec, vec, vec, full, vec,
        ],
        out_specs=pl.BlockSpec((_ROW_BLK, EMBED_DIM), lambda i: (i, 0)),
        out_shape=jax.ShapeDtypeStruct((nrows, EMBED_DIM), jnp.float32),
    )(e, A, b1c.reshape(1, EMBED_DIM), gamma.reshape(1, EMBED_DIM),
      beta.reshape(1, EMBED_DIM), W2.T, b2.reshape(1, EMBED_DIM))


@jax.jit
def kernel(task_id, table, W1, b1, gamma, beta, W2, b2):
    e = _make_sc_gather(BATCH)(table, task_id.astype(jnp.int32))
    return _mlp(e, W1, b1, gamma, beta, W2, b2)


# speedup vs baseline: 1.0089x; 1.0089x over previous
import functools

import jax
import jax.numpy as jnp
from jax import lax
from jax.experimental import pallas as pl
from jax.experimental.pallas import tpu as pltpu
from jax.experimental.pallas import tpu_sc as plsc

NUM_TASKS = 100000
EMBED_DIM = 128
BATCH = 16384

_NC = 2
_NS = 16
_NW = _NC * _NS
_B_PER_W = BATCH // _NW


@functools.lru_cache(maxsize=4)
def _make_sc_gather(nrows):
    b_per_w = nrows // _NW
    mesh = plsc.VectorSubcoreMesh(core_axis_name="c", subcore_axis_name="s")

    @functools.partial(
        pl.kernel,
        mesh=mesh,
        out_type=jax.ShapeDtypeStruct((nrows, EMBED_DIM), jnp.float32),
        scratch_types=[
            pltpu.VMEM((b_per_w,), jnp.int32),
            pltpu.VMEM((b_per_w, EMBED_DIM), jnp.float32),
            pltpu.SemaphoreType.DMA,
        ],
    )
    def sc_gather(table_hbm, idx_hbm, out_hbm, idx_v, rows_v, sem):
        wid = lax.axis_index("s") * _NC + lax.axis_index("c")
        base = wid * b_per_w
        pltpu.sync_copy(idx_hbm.at[pl.ds(base, b_per_w)], idx_v)
        pltpu.async_copy(table_hbm.at[idx_v], rows_v, sem).wait()
        pltpu.sync_copy(rows_v, out_hbm.at[pl.ds(base, b_per_w)])

    return sc_gather


_ROW_BLK = 4096


def _mlp_body(e_ref, a_ref, b1c_ref, gamma_ref, beta_ref, w2t_ref, b2_ref,
              out_ref):
    e = e_ref[...].astype(jnp.bfloat16)
    hc = lax.dot_general(e, a_ref[...], (((1,), (0,)), ((), ())),
                         preferred_element_type=jnp.float32)
    hc = hc + b1c_ref[...]
    avg = jnp.full((EMBED_DIM, EMBED_DIM), 1.0 / EMBED_DIM,
                   dtype=jnp.bfloat16)
    var = lax.dot_general((hc * hc).astype(jnp.bfloat16), avg,
                          (((1,), (0,)), ((), ())),
                          preferred_element_type=jnp.float32)
    h = hc * lax.rsqrt(var + 1e-5) * gamma_ref[...] + beta_ref[...]
    h = 0.5 * h * (1.0 + lax.erf(h * 0.7071067811865476))
    out = lax.dot_general(h.astype(jnp.bfloat16), w2t_ref[...],
                          (((1,), (0,)), ((), ())),
                          preferred_element_type=jnp.float32)
    out_ref[...] = out + b2_ref[...]


def _mlp(e, W1, b1, gamma, beta, W2, b2):
    A = (W1.T - jnp.mean(W1, axis=0)[:, None]).astype(jnp.bfloat16)
    b1c = b1 - jnp.mean(b1)
    nrows = e.shape[0]
    grid = (nrows // _ROW_BLK,)
    full = pl.BlockSpec((EMBED_DIM, EMBED_DIM), lambda i: (0, 0))
    vec = pl.BlockSpec((1, EMBED_DIM), lambda i: (0, 0))
    return pl.pallas_call(
        _mlp_body,
        grid=grid,
        in_specs=[
            pl.BlockSpec((_ROW_BLK, EMBED_DIM), lambda i: (i, 0)),
            full, vec, vec, vec, full, vec,
        ],
        out_specs=pl.BlockSpec((_ROW_BLK, EMBED_DIM), lambda i: (i, 0)),
        out_shape=jax.ShapeDtypeStruct((nrows, EMBED_DIM), jnp.float32),
    )(e, A, b1c.reshape(1, EMBED_DIM), gamma.reshape(1, EMBED_DIM),
      beta.reshape(1, EMBED_DIM), W2.T.astype(jnp.bfloat16),
      b2.reshape(1, EMBED_DIM))


@jax.jit
def kernel(task_id, table, W1, b1, gamma, beta, W2, b2):
    e = _make_sc_gather(BATCH)(table, task_id.astype(jnp.int32))
    return _mlp(e, W1, b1, gamma, beta, W2, b2)
